# Initial kernel scaffold; baseline (speedup 1.0000x reference)
import jax
import jax.numpy as jnp
from jax.experimental import pallas as pl


def kernel(x, table, W, b):
    raise NotImplementedError("write your pallas kernel here")


# speedup vs baseline: 97.2631x; 97.2631x over previous
import functools

import jax
import jax.numpy as jnp
from jax import lax
from jax.experimental import pallas as pl
from jax.experimental.pallas import tpu as pltpu
from jax.experimental.pallas import tpu_sc as plsc

NUM_EMB = 1000
EMB_DIM = 10
OUT_DIM = 2
BATCH = 16384
HIST = 200

NC = 2
NS = 16
L = 16
NW = NC * NS
ROWS_PER_W = BATCH // NW
GROUPS = ROWS_PER_W // L


def _fold_body(table_ref, w_ref, b_ref, t_ref):
    prod = lax.dot_general(
        w_ref[...], table_ref[...],
        (((1,), (1,)), ((), ())),
        preferred_element_type=jnp.float32,
    )
    t_ref[...] = prod * (1.0 / HIST) + b_ref[...].reshape(OUT_DIM, 1) * (1.0 / HIST)


def _fold_table(table, W, b):
    return pl.pallas_call(
        _fold_body,
        out_shape=jax.ShapeDtypeStruct((OUT_DIM, NUM_EMB), jnp.float32),
    )(table, W, b)


def _make_sc_kernel():
    mesh = plsc.VectorSubcoreMesh(
        core_axis_name="c", subcore_axis_name="s",
        num_cores=NC, num_subcores=NS,
    )

    @functools.partial(
        pl.kernel,
        out_type=jax.ShapeDtypeStruct((BATCH * OUT_DIM,), jnp.float32),
        mesh=mesh,
        compiler_params=pltpu.CompilerParams(needs_layout_passes=False),
        scratch_types=[
            pltpu.VMEM((OUT_DIM * NUM_EMB,), jnp.float32),
            pltpu.VMEM((L * HIST,), jnp.int32),
            pltpu.VMEM((ROWS_PER_W * OUT_DIM,), jnp.float32),
        ],
    )
    def sc_embed(x_hbm, t_hbm, out_hbm, t_v, x_v, out_v):
        wid = lax.axis_index("s") * NC + lax.axis_index("c")
        row0 = wid * ROWS_PER_W
        pltpu.sync_copy(t_hbm, t_v)

        riota = lax.iota(jnp.int32, L)
        xbase = riota * HIST
        dbase = riota * OUT_DIM

        def group_body(g, carry):
            pltpu.sync_copy(
                x_hbm.at[pl.ds((row0 + g * L) * HIST, L * HIST)], x_v)

            def l_body(l, accs):
                a0, a1, xoff = accs
                idx = plsc.load_gather(x_v, [xoff])
                v0 = plsc.load_gather(t_v, [idx])
                v1 = plsc.load_gather(t_v, [idx + NUM_EMB])
                return (a0 + v0, a1 + v1, xoff + 1)

            a0, a1, _ = lax.fori_loop(
                0, HIST, l_body,
                (jnp.zeros((L,), jnp.float32), jnp.zeros((L,), jnp.float32),
                 xbase),
            )
            didx = g * (L * OUT_DIM) + dbase
            plsc.store_scatter(out_v, [didx], a0)
            plsc.store_scatter(out_v, [didx + 1], a1)
            return carry

        lax.fori_loop(0, GROUPS, group_body, 0)
        pltpu.sync_copy(
            out_v, out_hbm.at[pl.ds(row0 * OUT_DIM, ROWS_PER_W * OUT_DIM)])

    return sc_embed


_sc_embed = _make_sc_kernel()


def kernel(x, table, W, b):
    t = _fold_table(table, W, b).reshape(OUT_DIM * NUM_EMB)
    out = _sc_embed(x.reshape(BATCH * HIST), t)
    return out.reshape(BATCH, OUT_DIM)


# speedup vs baseline: 157.4256x; 1.6186x over previous
import functools

import jax
import jax.numpy as jnp
from jax import lax
from jax.experimental import pallas as pl
from jax.experimental.pallas import tpu as pltpu
from jax.experimental.pallas import tpu_sc as plsc

NUM_EMB = 1000
EMB_DIM = 10
OUT_DIM = 2
BATCH = 16384
HIST = 200

NC = 2
NS = 16
L = 16
NW = NC * NS
ROWS_PER_W = BATCH // NW
GROUPS = ROWS_PER_W // L

U = 8
NACC = 4
CH = 4
NCH = GROUPS // CH
CHW = CH * L * HIST


def _fold_body(table_ref, w_ref, b_ref, pk_ref):
    prod = lax.dot_general(
        w_ref[...], table_ref[...],
        (((1,), (1,)), ((), ())),
        preferred_element_type=jnp.float32,
    )
    t = prod * (1.0 / HIST) + b_ref[...].reshape(OUT_DIM, 1) * (1.0 / HIST)
    bits = lax.bitcast_convert_type(t, jnp.uint32)
    rnd = bits + jnp.uint32(0x7FFF) + ((bits >> 16) & jnp.uint32(1))
    top = rnd & jnp.uint32(0xFFFF0000)
    pk = top[1:2, :] | (top[0:1, :] >> 16)
    pk_ref[...] = lax.bitcast_convert_type(pk, jnp.int32)


def _fold_table(table, W, b):
    return pl.pallas_call(
        _fold_body,
        out_shape=jax.ShapeDtypeStruct((1, NUM_EMB), jnp.int32),
    )(table, W, b)


def _make_sc_kernel():
    mesh = plsc.VectorSubcoreMesh(
        core_axis_name="c", subcore_axis_name="s",
        num_cores=NC, num_subcores=NS,
    )

    @functools.partial(
        pl.kernel,
        out_type=jax.ShapeDtypeStruct((BATCH * OUT_DIM,), jnp.float32),
        mesh=mesh,
        compiler_params=pltpu.CompilerParams(needs_layout_passes=False),
        scratch_types=[
            pltpu.VMEM((NUM_EMB,), jnp.int32),
            pltpu.VMEM((2 * CHW,), jnp.int32),
            pltpu.VMEM((ROWS_PER_W * OUT_DIM,), jnp.float32),
            pltpu.SemaphoreType.DMA,
            pltpu.SemaphoreType.DMA,
        ],
    )
    def sc_embed(x_hbm, t_hbm, out_hbm, t_v, x_v, out_v, sem0, sem1):
        wid = lax.axis_index("s") * NC + lax.axis_index("c")
        row0 = wid * ROWS_PER_W
        xflat0 = row0 * HIST
        pltpu.sync_copy(t_hbm, t_v)

        riota = lax.iota(jnp.int32, L)
        rbase = riota * HIST
        dbase = riota * OUT_DIM
        sems = (sem0, sem1)
        mask_hi = jnp.int32(-65536)

        def chunk_src(c):
            return x_hbm.at[pl.ds(xflat0 + c * CHW, CHW)]

        def buf_dst(buf):
            return x_v.at[pl.ds(buf * CHW, CHW)]

        pending = [pltpu.async_copy(chunk_src(0), buf_dst(0), sem0), None]
        for c in range(NCH):
            buf = c & 1
            pending[buf].wait()
            if c + 1 < NCH:
                nb = 1 - buf
                pending[nb] = pltpu.async_copy(
                    chunk_src(c + 1), buf_dst(nb), sems[nb])

            def group_body(g, carry, *, _buf=buf, _c=c):
                xoff0 = _buf * CHW + g * (L * HIST) + rbase

                def l_body(i, inner):
                    *accs, xoff = inner
                    accs = list(accs)
                    for k in range(U):
                        idx = plsc.load_gather(x_v, [xoff + k if k else xoff])
                        w = plsc.load_gather(t_v, [idx])
                        v1 = plsc.bitcast(w & mask_hi, jnp.float32)
                        v0 = plsc.bitcast(w << 16, jnp.float32)
                        j = k % NACC
                        accs[2 * j] = accs[2 * j] + v0
                        accs[2 * j + 1] = accs[2 * j + 1] + v1
                    return (*accs, xoff + U)

                z = jnp.zeros((L,), jnp.float32)
                res = lax.fori_loop(
                    0, HIST // U, l_body, ((z,) * (2 * NACC)) + (xoff0,))
                a0 = (res[0] + res[2]) + (res[4] + res[6])
                a1 = (res[1] + res[3]) + (res[5] + res[7])
                didx = (_c * CH + g) * (L * OUT_DIM) + dbase
                plsc.store_scatter(out_v, [didx], a0)
                plsc.store_scatter(out_v, [didx + 1], a1)
                return carry

            lax.fori_loop(0, CH, group_body, 0)

        pltpu.sync_copy(
            out_v, out_hbm.at[pl.ds(row0 * OUT_DIM, ROWS_PER_W * OUT_DIM)])

    return sc_embed


_sc_embed = _make_sc_kernel()


def kernel(x, table, W, b):
    t = _fold_table(table, W, b).reshape(NUM_EMB)
    out = _sc_embed(x.reshape(BATCH * HIST), t)
    return out.reshape(BATCH, OUT_DIM)
